# baseline (speedup 1.0000x reference)
import functools

import jax
import jax.numpy as jnp
from jax.experimental import pallas as pl
from jax.experimental.pallas import tpu as pltpu

B, S, K, NDIM, NPROP, NFILT = 8, 512, 64, 8, 128, 196
NLAYERS = 4
_HI = jax.lax.Precision.HIGHEST


def _elu(x):
    return jnp.where(x > 0, x, jnp.exp(jnp.minimum(x, 0.0)) - 1.0)


def _mm(a, b):
    return jax.lax.dot_general(a, b, (((1,), (0,)), ((), ())),
                               preferred_element_type=jnp.float32,
                               precision=_HI)


def _tn(a, b):
    return jax.lax.dot_general(a, b, (((0,), (0,)), ((), ())),
                               preferred_element_type=jnp.float32,
                               precision=_HI)


def _kth_smallest_bits_cols(vi, k):
    v16 = vi >> 16
    lo = jnp.zeros((1, S), jnp.int32)
    hi = jnp.full((1, S), 0x7F80, jnp.int32)
    for _ in range(15):
        mid = lo + ((hi - lo) >> 1)
        cnt = jnp.sum((v16 <= mid).astype(jnp.int32), axis=0, keepdims=True)
        pred = cnt >= k
        hi = jnp.where(pred, mid, hi)
        lo = jnp.where(pred, lo, mid + 1)
    return ((hi + 1) << 16) - 1


_GRP = 128


def _max_product2(wm_ref, fs, fmax_ref, eye_g):
    _CH = 128
    fbs = [f.astype(jnp.bfloat16) for f in fs]

    def body(g, carry):
        for h, fb in enumerate(fbs):
            w = wm_ref[pl.ds(h * S + g * _GRP, _GRP), :]
            wt = _tn(w, eye_g).astype(jnp.bfloat16)
            acc = [None] * _GRP
            for c in range(S // _CH):
                fbc = fb[c * _CH:(c + 1) * _CH, :]
                wtc = wt[c * _CH:(c + 1) * _CH, :]
                for ii in range(_GRP):
                    part = jnp.max(wtc[:, ii:ii + 1] * fbc, axis=0,
                                   keepdims=True)
                    acc[ii] = part if c == 0 else jnp.maximum(acc[ii], part)
            fmax_ref[pl.ds(h * S + g * _GRP, _GRP), :] = jnp.concatenate(
                acc, axis=0).astype(jnp.float32)
        return carry

    jax.lax.fori_loop(0, S // _GRP, body, 0)


def _seg_kernel(*refs):
    (feat_ref, *prefs), o_ref, wm_ref, fmax_ref = refs[:-3], refs[-3], refs[-2], refs[-1]
    params = list(prefs)

    eye = (jax.lax.broadcasted_iota(jnp.int32, (S, S), 0)
           == jax.lax.broadcasted_iota(jnp.int32, (S, S), 1))
    eye_g = (jax.lax.broadcasted_iota(jnp.int32, (_GRP, _GRP), 0)
             == jax.lax.broadcasted_iota(jnp.int32, (_GRP, _GRP), 1)
             ).astype(jnp.float32)
    ones11 = jnp.ones((1, 1), jnp.float32)

    nxt = iter(params)

    def take(n):
        return [next(nxt) for _ in range(n)]

    xs, feats_l = [], []
    for h in range(2):
        x_basic = feat_ref[h * S:(h + 1) * S, :]
        m = jnp.mean(x_basic, axis=0, keepdims=True)
        xs.append(jnp.concatenate(
            [x_basic, jnp.broadcast_to(m, x_basic.shape)], axis=1))
        feats_l.append([x_basic])

    d1w, d1b = take(2)
    for h in range(2):
        xs[h] = _elu(_mm(xs[h], d1w[...]) + d1b[...])
        feats_l[h].append(xs[h])

    for _ in range(NLAYERS):
        spw, spb, ftw, ftb, outw, outb, w1, b1, w2, b2, w3, b3 = take(12)
        fs, fmeans = [], []
        for h in range(2):
            x = xs[h]
            coords = _mm(x, spw[...]) + spb[...]
            f = jax.nn.relu(_mm(x, ftw[...]) + ftb[...])
            c2 = coords * coords
            n2 = jnp.sum(c2, axis=1, keepdims=True)
            n2t = jax.lax.dot_general(
                jnp.ones((1, NDIM), jnp.float32), c2, (((1,), (1,)), ((), ())),
                preferred_element_type=jnp.float32, precision=_HI)
            g = jax.lax.dot_general(coords, coords, (((1,), (1,)), ((), ())),
                                    preferred_element_type=jnp.float32,
                                    precision=_HI)
            d2 = n2 + n2t - 2.0 * g
            d2 = jnp.where(eye, 0.0, jnp.maximum(d2, 0.0))
            vi = jax.lax.bitcast_convert_type(d2, jnp.int32)
            tt = _kth_smallest_bits_cols(vi, K)
            tf_row = jax.lax.bitcast_convert_type(tt, jnp.float32)
            tf_col = _tn(tf_row, ones11)
            wm = jnp.where(d2 <= tf_col, jnp.exp(-10.0 * d2), 0.0)
            fmeans.append(_mm(wm, f) * (1.0 / K))
            fs.append(f)
            wm_ref[h * S:(h + 1) * S, :] = wm
        _max_product2(wm_ref, fs, fmax_ref, eye_g)
        for h in range(2):
            fmax = fmax_ref[h * S:(h + 1) * S, :]
            x = jax.nn.relu(
                _mm(jnp.concatenate([xs[h], fmeans[h], fmax], axis=1),
                    outw[...]) + outb[...])
            x = jax.nn.relu(_mm(x, w1[...]) + b1[...])
            x = jax.nn.relu(_mm(x, w2[...]) + b2[...])
            mm_ = jnp.mean(x, axis=0, keepdims=True)
            x = jnp.concatenate([x, jnp.broadcast_to(mm_, x.shape)], axis=1)
            x = jax.nn.relu(_mm(x, w3[...]) + b3[...])
            feats_l[h].append(x)
            xs[h] = x

    ow1, ob1, ow2, ob2, ow3, ob3, ow4, ob4, ow5, ob5 = take(10)
    for h in range(2):
        x = jnp.concatenate(feats_l[h], axis=1)
        x = jax.nn.relu(_mm(x, ow1[...]) + ob1[...])
        x = _elu(_mm(x, ow2[...]) + ob2[...])
        x = _elu(_mm(x, ow3[...]) + ob3[...])
        x = jax.nn.relu(_mm(x, ow4[...]) + ob4[...])
        x = jax.nn.relu(_mm(x, ow5[...]) + ob5[...])
        o_ref[h * S:(h + 1) * S, :] = x


def _param_order():
    names = ['dense1_W', 'dense1_b']
    for i in range(NLAYERS):
        names += [f'gn{i}_sp_W', f'gn{i}_sp_b', f'gn{i}_ft_W', f'gn{i}_ft_b',
                  f'gn{i}_out_W', f'gn{i}_out_b',
                  f'b{i}_dn1_W', f'b{i}_dn1_b', f'b{i}_dn2_W', f'b{i}_dn2_b',
                  f'b{i}_dn3_W', f'b{i}_dn3_b']
    for j in range(1, 6):
        names += [f'odn{j}_W', f'odn{j}_b']
    return names


@functools.partial(jax.jit, static_argnames=())
def kernel(feat, row_splits, test_arr, params):
    plist = []
    for name in _param_order():
        a = params[name]
        if a.ndim == 1:
            a = a.reshape(1, -1)
        plist.append(a)

    in_specs = [pl.BlockSpec((2 * S, 64), lambda i: (i, 0))]
    for a in plist:
        in_specs.append(pl.BlockSpec(a.shape, lambda i: (0, 0)))

    out = pl.pallas_call(
        _seg_kernel,
        grid=(B // 2,),
        in_specs=in_specs,
        out_specs=pl.BlockSpec((2 * S, 128), lambda i: (i, 0)),
        out_shape=jax.ShapeDtypeStruct((B * S, 128), jnp.float32),
        scratch_shapes=[pltpu.VMEM((2 * S, S), jnp.float32),
                        pltpu.VMEM((2 * S, NPROP), jnp.float32)],
        compiler_params=pltpu.CompilerParams(
            dimension_semantics=("parallel",)),
    )(feat, *plist)
    return out


# speedup vs baseline: 1.0536x; 1.0536x over previous
import functools

import jax
import jax.numpy as jnp
from jax.experimental import pallas as pl
from jax.experimental.pallas import tpu as pltpu

B, S, K, NDIM, NPROP, NFILT = 8, 512, 64, 8, 128, 196
NLAYERS = 4
_HI = jax.lax.Precision.HIGHEST


def _elu(x):
    return jnp.where(x > 0, x, jnp.exp(jnp.minimum(x, 0.0)) - 1.0)


def _mm(a, b):
    return jax.lax.dot_general(a, b, (((1,), (0,)), ((), ())),
                               preferred_element_type=jnp.float32,
                               precision=_HI)


def _tn(a, b):
    return jax.lax.dot_general(a, b, (((0,), (0,)), ((), ())),
                               preferred_element_type=jnp.float32,
                               precision=_HI)


def _kth_smallest_bits_cols(vi, k):
    v16 = vi >> 16
    lo = jnp.zeros((1, S), jnp.int32)
    hi = jnp.full((1, S), 0x7F80, jnp.int32)
    for _ in range(15):
        mid = lo + ((hi - lo) >> 1)
        cnt = jnp.sum((v16 <= mid).astype(jnp.int32), axis=0, keepdims=True)
        pred = cnt >= k
        hi = jnp.where(pred, mid, hi)
        lo = jnp.where(pred, lo, mid + 1)
    return ((hi + 1) << 16) - 1


_GRP = 256


def _max_product2(wm_ref, fs, fmax_ref, eye_g):
    fbs = [f.astype(jnp.bfloat16) for f in fs]

    def body(g, carry):
        for h, fb in enumerate(fbs):
            w = wm_ref[pl.ds(h * S + g * _GRP, _GRP), :]
            wt = _tn(w, eye_g).astype(jnp.bfloat16)
            rows = [jnp.max(wt[:, ii:ii + 1] * fb, axis=0, keepdims=True)
                    for ii in range(_GRP)]
            fmax_ref[pl.ds(h * S + g * _GRP, _GRP), :] = jnp.concatenate(
                rows, axis=0).astype(jnp.float32)
        return carry

    jax.lax.fori_loop(0, S // _GRP, body, 0)


def _seg_kernel(*refs):
    (feat_ref, *prefs), o_ref, wm_ref, fmax_ref = refs[:-3], refs[-3], refs[-2], refs[-1]
    params = list(prefs)

    eye = (jax.lax.broadcasted_iota(jnp.int32, (S, S), 0)
           == jax.lax.broadcasted_iota(jnp.int32, (S, S), 1))
    eye_g = (jax.lax.broadcasted_iota(jnp.int32, (_GRP, _GRP), 0)
             == jax.lax.broadcasted_iota(jnp.int32, (_GRP, _GRP), 1)
             ).astype(jnp.float32)
    ones11 = jnp.ones((1, 1), jnp.float32)

    nxt = iter(params)

    def take(n):
        return [next(nxt) for _ in range(n)]

    xs, feats_l = [], []
    for h in range(2):
        x_basic = feat_ref[h * S:(h + 1) * S, :]
        m = jnp.mean(x_basic, axis=0, keepdims=True)
        xs.append(jnp.concatenate(
            [x_basic, jnp.broadcast_to(m, x_basic.shape)], axis=1))
        feats_l.append([x_basic])

    d1w, d1b = take(2)
    for h in range(2):
        xs[h] = _elu(_mm(xs[h], d1w[...]) + d1b[...])
        feats_l[h].append(xs[h])

    for _ in range(NLAYERS):
        spw, spb, ftw, ftb, outw, outb, w1, b1, w2, b2, w3, b3 = take(12)
        fs, fmeans = [], []
        for h in range(2):
            x = xs[h]
            coords = _mm(x, spw[...]) + spb[...]
            f = jax.nn.relu(_mm(x, ftw[...]) + ftb[...])
            c2 = coords * coords
            n2 = jnp.sum(c2, axis=1, keepdims=True)
            n2t = jax.lax.dot_general(
                jnp.ones((1, NDIM), jnp.float32), c2, (((1,), (1,)), ((), ())),
                preferred_element_type=jnp.float32, precision=_HI)
            g = jax.lax.dot_general(coords, coords, (((1,), (1,)), ((), ())),
                                    preferred_element_type=jnp.float32,
                                    precision=_HI)
            d2 = n2 + n2t - 2.0 * g
            d2 = jnp.where(eye, 0.0, jnp.maximum(d2, 0.0))
            vi = jax.lax.bitcast_convert_type(d2, jnp.int32)
            tt = _kth_smallest_bits_cols(vi, K)
            tf_row = jax.lax.bitcast_convert_type(tt, jnp.float32)
            tf_col = _tn(tf_row, ones11)
            wm = jnp.where(d2 <= tf_col, jnp.exp(-10.0 * d2), 0.0)
            fmeans.append(_mm(wm, f) * (1.0 / K))
            fs.append(f)
            wm_ref[h * S:(h + 1) * S, :] = wm
        _max_product2(wm_ref, fs, fmax_ref, eye_g)
        for h in range(2):
            fmax = fmax_ref[h * S:(h + 1) * S, :]
            x = jax.nn.relu(
                _mm(jnp.concatenate([xs[h], fmeans[h], fmax], axis=1),
                    outw[...]) + outb[...])
            x = jax.nn.relu(_mm(x, w1[...]) + b1[...])
            x = jax.nn.relu(_mm(x, w2[...]) + b2[...])
            mm_ = jnp.mean(x, axis=0, keepdims=True)
            x = jnp.concatenate([x, jnp.broadcast_to(mm_, x.shape)], axis=1)
            x = jax.nn.relu(_mm(x, w3[...]) + b3[...])
            feats_l[h].append(x)
            xs[h] = x

    ow1, ob1, ow2, ob2, ow3, ob3, ow4, ob4, ow5, ob5 = take(10)
    for h in range(2):
        x = jnp.concatenate(feats_l[h], axis=1)
        x = jax.nn.relu(_mm(x, ow1[...]) + ob1[...])
        x = _elu(_mm(x, ow2[...]) + ob2[...])
        x = _elu(_mm(x, ow3[...]) + ob3[...])
        x = jax.nn.relu(_mm(x, ow4[...]) + ob4[...])
        x = jax.nn.relu(_mm(x, ow5[...]) + ob5[...])
        o_ref[h * S:(h + 1) * S, :] = x


def _param_order():
    names = ['dense1_W', 'dense1_b']
    for i in range(NLAYERS):
        names += [f'gn{i}_sp_W', f'gn{i}_sp_b', f'gn{i}_ft_W', f'gn{i}_ft_b',
                  f'gn{i}_out_W', f'gn{i}_out_b',
                  f'b{i}_dn1_W', f'b{i}_dn1_b', f'b{i}_dn2_W', f'b{i}_dn2_b',
                  f'b{i}_dn3_W', f'b{i}_dn3_b']
    for j in range(1, 6):
        names += [f'odn{j}_W', f'odn{j}_b']
    return names


@functools.partial(jax.jit, static_argnames=())
def kernel(feat, row_splits, test_arr, params):
    plist = []
    for name in _param_order():
        a = params[name]
        if a.ndim == 1:
            a = a.reshape(1, -1)
        plist.append(a)

    in_specs = [pl.BlockSpec((2 * S, 64), lambda i: (i, 0))]
    for a in plist:
        in_specs.append(pl.BlockSpec(a.shape, lambda i: (0, 0)))

    out = pl.pallas_call(
        _seg_kernel,
        grid=(B // 2,),
        in_specs=in_specs,
        out_specs=pl.BlockSpec((2 * S, 128), lambda i: (i, 0)),
        out_shape=jax.ShapeDtypeStruct((B * S, 128), jnp.float32),
        scratch_shapes=[pltpu.VMEM((2 * S, S), jnp.float32),
                        pltpu.VMEM((2 * S, NPROP), jnp.float32)],
        compiler_params=pltpu.CompilerParams(
            dimension_semantics=("parallel",)),
    )(feat, *plist)
    return out


# speedup vs baseline: 1.1542x; 1.0954x over previous
import functools

import jax
import jax.numpy as jnp
from jax.experimental import pallas as pl
from jax.experimental.pallas import tpu as pltpu

B, S, K, NDIM, NPROP, NFILT = 8, 512, 64, 8, 128, 196
NLAYERS = 4
_HI = jax.lax.Precision.HIGHEST


def _elu(x):
    return jnp.where(x > 0, x, jnp.exp(jnp.minimum(x, 0.0)) - 1.0)


def _mm(a, b):
    return jax.lax.dot_general(a, b, (((1,), (0,)), ((), ())),
                               preferred_element_type=jnp.float32,
                               precision=_HI)


def _tn(a, b, precision=_HI):
    return jax.lax.dot_general(a, b, (((0,), (0,)), ((), ())),
                               preferred_element_type=jnp.float32,
                               precision=precision)


def _kth_smallest_bits_cols(vi, k):
    v16 = vi >> 16
    lo = jnp.zeros((1, S), jnp.int32)
    hi = jnp.full((1, S), 0x7F80, jnp.int32)
    for _ in range(15):
        mid = lo + ((hi - lo) >> 1)
        cnt = jnp.sum((v16 <= mid).astype(jnp.int32), axis=0, keepdims=True)
        pred = cnt >= k
        hi = jnp.where(pred, mid, hi)
        lo = jnp.where(pred, lo, mid + 1)
    return ((hi + 1) << 16) - 1


_GRP = 128


def _max_product2(wm_ref, fs, fmax_ref, eye_g):
    fbs = [f.astype(jnp.bfloat16) for f in fs]

    def body(g, carry):
        for h, fb in enumerate(fbs):
            w = wm_ref[pl.ds(h * S + g * _GRP, _GRP), :]
            wt = _tn(w, eye_g,
                     jax.lax.Precision.DEFAULT).astype(jnp.bfloat16)
            rows = [jnp.max(wt[:, ii:ii + 1] * fb, axis=0, keepdims=True)
                    for ii in range(_GRP)]
            fmax_ref[pl.ds(h * S + g * _GRP, _GRP), :] = jnp.concatenate(
                rows, axis=0).astype(jnp.float32)
        return carry

    jax.lax.fori_loop(0, S // _GRP, body, 0)


def _seg_kernel(*refs):
    (feat_ref, *prefs), o_ref, wm_ref, fmax_ref = refs[:-3], refs[-3], refs[-2], refs[-1]
    params = list(prefs)

    eye = (jax.lax.broadcasted_iota(jnp.int32, (S, S), 0)
           == jax.lax.broadcasted_iota(jnp.int32, (S, S), 1))
    eye_g = (jax.lax.broadcasted_iota(jnp.int32, (_GRP, _GRP), 0)
             == jax.lax.broadcasted_iota(jnp.int32, (_GRP, _GRP), 1)
             ).astype(jnp.float32)
    ones11 = jnp.ones((1, 1), jnp.float32)

    nxt = iter(params)

    def take(n):
        return [next(nxt) for _ in range(n)]

    xs, feats_l = [], []
    for h in range(2):
        x_basic = feat_ref[h * S:(h + 1) * S, :]
        m = jnp.mean(x_basic, axis=0, keepdims=True)
        xs.append(jnp.concatenate(
            [x_basic, jnp.broadcast_to(m, x_basic.shape)], axis=1))
        feats_l.append([x_basic])

    d1w, d1b = take(2)
    for h in range(2):
        xs[h] = _elu(_mm(xs[h], d1w[...]) + d1b[...])
        feats_l[h].append(xs[h])

    for _ in range(NLAYERS):
        spw, spb, ftw, ftb, outw, outb, w1, b1, w2, b2, w3, b3 = take(12)
        fs, fmeans = [], []
        for h in range(2):
            x = xs[h]
            coords = _mm(x, spw[...]) + spb[...]
            f = jax.nn.relu(_mm(x, ftw[...]) + ftb[...])
            c2 = coords * coords
            n2 = jnp.sum(c2, axis=1, keepdims=True)
            n2t = jax.lax.dot_general(
                jnp.ones((1, NDIM), jnp.float32), c2, (((1,), (1,)), ((), ())),
                preferred_element_type=jnp.float32, precision=_HI)
            g = jax.lax.dot_general(coords, coords, (((1,), (1,)), ((), ())),
                                    preferred_element_type=jnp.float32,
                                    precision=_HI)
            d2 = n2 + n2t - 2.0 * g
            d2 = jnp.where(eye, 0.0, jnp.maximum(d2, 0.0))
            vi = jax.lax.bitcast_convert_type(d2, jnp.int32)
            tt = _kth_smallest_bits_cols(vi, K)
            tf_row = jax.lax.bitcast_convert_type(tt, jnp.float32)
            tf_col = _tn(tf_row, ones11)
            wm = jnp.where(d2 <= tf_col, jnp.exp(-10.0 * d2), 0.0)
            fmeans.append(jax.lax.dot_general(
                wm, f, (((1,), (0,)), ((), ())),
                preferred_element_type=jnp.float32,
                precision=jax.lax.Precision.DEFAULT) * (1.0 / K))
            fs.append(f)
            wm_ref[h * S:(h + 1) * S, :] = wm
        _max_product2(wm_ref, fs, fmax_ref, eye_g)
        for h in range(2):
            fmax = fmax_ref[h * S:(h + 1) * S, :]
            x = jax.nn.relu(
                _mm(jnp.concatenate([xs[h], fmeans[h], fmax], axis=1),
                    outw[...]) + outb[...])
            x = jax.nn.relu(_mm(x, w1[...]) + b1[...])
            x = jax.nn.relu(_mm(x, w2[...]) + b2[...])
            mm_ = jnp.mean(x, axis=0, keepdims=True)
            x = jnp.concatenate([x, jnp.broadcast_to(mm_, x.shape)], axis=1)
            x = jax.nn.relu(_mm(x, w3[...]) + b3[...])
            feats_l[h].append(x)
            xs[h] = x

    ow1, ob1, ow2, ob2, ow3, ob3, ow4, ob4, ow5, ob5 = take(10)
    for h in range(2):
        x = jnp.concatenate(feats_l[h], axis=1)
        x = jax.nn.relu(_mm(x, ow1[...]) + ob1[...])
        x = _elu(_mm(x, ow2[...]) + ob2[...])
        x = _elu(_mm(x, ow3[...]) + ob3[...])
        x = jax.nn.relu(_mm(x, ow4[...]) + ob4[...])
        x = jax.nn.relu(_mm(x, ow5[...]) + ob5[...])
        o_ref[h * S:(h + 1) * S, :] = x


def _param_order():
    names = ['dense1_W', 'dense1_b']
    for i in range(NLAYERS):
        names += [f'gn{i}_sp_W', f'gn{i}_sp_b', f'gn{i}_ft_W', f'gn{i}_ft_b',
                  f'gn{i}_out_W', f'gn{i}_out_b',
                  f'b{i}_dn1_W', f'b{i}_dn1_b', f'b{i}_dn2_W', f'b{i}_dn2_b',
                  f'b{i}_dn3_W', f'b{i}_dn3_b']
    for j in range(1, 6):
        names += [f'odn{j}_W', f'odn{j}_b']
    return names


@functools.partial(jax.jit, static_argnames=())
def kernel(feat, row_splits, test_arr, params):
    plist = []
    for name in _param_order():
        a = params[name]
        if a.ndim == 1:
            a = a.reshape(1, -1)
        plist.append(a)

    in_specs = [pl.BlockSpec((2 * S, 64), lambda i: (i, 0))]
    for a in plist:
        in_specs.append(pl.BlockSpec(a.shape, lambda i: (0, 0)))

    out = pl.pallas_call(
        _seg_kernel,
        grid=(B // 2,),
        in_specs=in_specs,
        out_specs=pl.BlockSpec((2 * S, 128), lambda i: (i, 0)),
        out_shape=jax.ShapeDtypeStruct((B * S, 128), jnp.float32),
        scratch_shapes=[pltpu.VMEM((2 * S, S), jnp.float32),
                        pltpu.VMEM((2 * S, NPROP), jnp.float32)],
        compiler_params=pltpu.CompilerParams(
            dimension_semantics=("parallel",)),
    )(feat, *plist)
    return out
